# baseline (speedup 1.0000x reference)
import jax
import jax.numpy as jnp
from jax.experimental import pallas as pl
from jax.experimental.pallas import tpu as pltpu

B, N, IN_F = 2, 4096, 128
H, D = 4, 32
K = 32
R = 512


def _prologue_kernel(x_ref, w_ref, asel_ref, eind_ref, hwe_ref):
    x = x_ref[0]
    W = w_ref[...]
    h = jnp.dot(x.astype(jnp.bfloat16), W.astype(jnp.bfloat16),
                preferred_element_type=jnp.float32)
    e = jnp.dot(h, asel_ref[...], precision=jax.lax.Precision.HIGHEST)
    f = jnp.where(e >= 0.0, e, 0.2 * e)
    fmax = jnp.max(f, axis=0, keepdims=True)
    expf = jnp.exp(f - fmax)
    expfb = jnp.dot(expf, eind_ref[...],
                    precision=jax.lax.Precision.HIGHEST)
    hwe_ref[0, :, : H * D] = h * expfb
    hwe_ref[0, :, H * D:] = expfb


def _main_kernel(pos_ref, post_ref, p2row_ref, hwe_ref, x_ref, lns_ref,
                 lnb_ref, out_ref):
    pos = pos_ref[0]
    posT = post_ref[0]
    dot = jnp.dot(pos.astype(jnp.bfloat16), posT.astype(jnp.bfloat16),
                  preferred_element_type=jnp.float32)
    p2b = jnp.sum(pos * pos, axis=1, keepdims=True)
    p2f = p2row_ref[0]
    d2 = p2b + p2f - 2.0 * dot

    def body(_, m):
        return jnp.min(jnp.where(d2 > m, d2, jnp.inf), axis=1, keepdims=True)

    m0 = jnp.full((R, 1), -jnp.inf, dtype=jnp.float32)
    tau = jax.lax.fori_loop(0, K, body, m0)

    def backoff(_, carry):
        tau, cnt = carry
        prev = jnp.max(jnp.where(d2 < tau, d2, -jnp.inf), axis=1,
                       keepdims=True)
        cntp = jnp.sum((d2 <= prev).astype(jnp.float32), axis=1,
                       keepdims=True)
        move = jnp.logical_and(cnt > K, cntp >= K)
        return jnp.where(move, prev, tau), jnp.where(move, cntp, cnt)

    cnt0 = jnp.sum((d2 <= tau).astype(jnp.float32), axis=1, keepdims=True)
    tau, _ = jax.lax.fori_loop(0, 4, backoff, (tau, cnt0))

    cntb = jnp.sum((d2 < tau).astype(jnp.float32), axis=1, keepdims=True)
    iota = jax.lax.broadcasted_iota(jnp.int32, (1, N), 1).astype(jnp.float32)
    jsel = jnp.where(d2 == tau, iota, jnp.inf)

    def tie_body(_, carry):
        jthr, cntj = carry
        jnext = jnp.min(jnp.where(jsel > jthr, jsel, jnp.inf), axis=1,
                        keepdims=True)
        move = cntj < K
        return (jnp.where(move, jnext, jthr),
                cntj + move.astype(jnp.float32))

    jthr0 = jnp.full((R, 1), -jnp.inf, dtype=jnp.float32)
    jthr, cntj = jax.lax.fori_loop(0, 4, tie_body, (jthr0, cntb))
    keep_all = cntj < K
    A = (jnp.logical_or(d2 < tau,
                        jnp.logical_or(jsel <= jthr,
                                       jnp.logical_and(d2 == tau, keep_all)))
         ).astype(jnp.float32)

    nm = jnp.dot(A, hwe_ref[0], precision=jax.lax.Precision.HIGHEST)
    hp = nm[:, : H * D] / nm[:, H * D:]
    y = hp + x_ref[0]
    mu = jnp.mean(y, axis=1, keepdims=True)
    yc = y - mu
    var = jnp.mean(yc * yc, axis=1, keepdims=True)
    out = yc * jax.lax.rsqrt(var + 1e-5) * lns_ref[...] + lnb_ref[...]
    out_ref[0] = out


def kernel(x, positions, W, a_src, a_dst, ln_scale, ln_bias, topk):
    HD = H * D
    a_flat = (a_src + a_dst).reshape(HD)
    grp = jnp.arange(HD, dtype=jnp.int32) // D
    heads = jnp.arange(H, dtype=jnp.int32)
    asel = jnp.where(grp[:, None] == heads[None, :], a_flat[:, None], 0.0)
    eind = (grp[None, :] == heads[:, None]).astype(jnp.float32)
    posT = jnp.transpose(positions, (0, 2, 1))
    p2row = ((positions ** 2).sum(axis=-1)).reshape(B, 1, N)
    lns = ln_scale.reshape(1, HD)
    lnb = ln_bias.reshape(1, HD)

    hwe = pl.pallas_call(
        _prologue_kernel,
        grid=(B,),
        in_specs=[
            pl.BlockSpec((1, N, IN_F), lambda b: (b, 0, 0)),
            pl.BlockSpec((IN_F, HD), lambda b: (0, 0)),
            pl.BlockSpec((HD, H), lambda b: (0, 0)),
            pl.BlockSpec((H, HD), lambda b: (0, 0)),
        ],
        out_specs=pl.BlockSpec((1, N, 2 * HD), lambda b: (b, 0, 0)),
        out_shape=jax.ShapeDtypeStruct((B, N, 2 * HD), jnp.float32),
        compiler_params=pltpu.CompilerParams(
            dimension_semantics=("parallel",)),
    )(x, W, asel, eind)

    out = pl.pallas_call(
        _main_kernel,
        grid=(B, N // R),
        in_specs=[
            pl.BlockSpec((1, R, 3), lambda b, i: (b, i, 0)),
            pl.BlockSpec((1, 3, N), lambda b, i: (b, 0, 0)),
            pl.BlockSpec((1, 1, N), lambda b, i: (b, 0, 0)),
            pl.BlockSpec((1, N, 2 * HD), lambda b, i: (b, 0, 0)),
            pl.BlockSpec((1, R, IN_F), lambda b, i: (b, i, 0)),
            pl.BlockSpec((1, HD), lambda b, i: (0, 0)),
            pl.BlockSpec((1, HD), lambda b, i: (0, 0)),
        ],
        out_specs=pl.BlockSpec((1, R, IN_F), lambda b, i: (b, i, 0)),
        out_shape=jax.ShapeDtypeStruct((B, N, IN_F), jnp.float32),
        compiler_params=pltpu.CompilerParams(
            dimension_semantics=("parallel", "parallel")),
    )(positions, posT, p2row, hwe, x, lns, lnb)
    return out


# speedup vs baseline: 1.0478x; 1.0478x over previous
import jax
import jax.numpy as jnp
from jax.experimental import pallas as pl
from jax.experimental.pallas import tpu as pltpu

B, N, IN_F = 2, 4096, 128
H, D = 4, 32
K = 32
R = 512


def _prologue_kernel(x_ref, w_ref, asel_ref, eind_ref, hwe_ref):
    x = x_ref[0]
    W = w_ref[...]
    h = jnp.dot(x.astype(jnp.bfloat16), W.astype(jnp.bfloat16),
                preferred_element_type=jnp.float32)
    e = jnp.dot(h, asel_ref[...], precision=jax.lax.Precision.HIGHEST)
    f = jnp.where(e >= 0.0, e, 0.2 * e)
    fmax = jnp.max(f, axis=0, keepdims=True)
    expf = jnp.exp(f - fmax)
    expfb = jnp.dot(expf, eind_ref[...],
                    precision=jax.lax.Precision.HIGHEST)
    hwe_ref[0, :, : H * D] = h * expfb
    hwe_ref[0, :, H * D:] = expfb


def _main_kernel(pos_ref, post_ref, p2row_ref, hwe_ref, x_ref, lns_ref,
                 lnb_ref, out_ref):
    pos = pos_ref[0]
    posT = post_ref[0]
    dot = jnp.dot(pos.astype(jnp.bfloat16), posT.astype(jnp.bfloat16),
                  preferred_element_type=jnp.float32)
    p2b = jnp.sum(pos * pos, axis=1, keepdims=True)
    p2f = p2row_ref[0]
    d2 = p2b + p2f - 2.0 * dot

    def body(_, m):
        return jnp.min(jnp.where(d2 > m, d2, jnp.inf), axis=1, keepdims=True)

    m0 = jnp.full((R, 1), -jnp.inf, dtype=jnp.float32)
    tau = jax.lax.fori_loop(0, K, body, m0, unroll=4)

    def backoff(_, carry):
        tau, cnt = carry
        prev = jnp.max(jnp.where(d2 < tau, d2, -jnp.inf), axis=1,
                       keepdims=True)
        cntp = jnp.sum((d2 <= prev).astype(jnp.float32), axis=1,
                       keepdims=True)
        move = jnp.logical_and(cnt > K, cntp >= K)
        return jnp.where(move, prev, tau), jnp.where(move, cntp, cnt)

    cnt0 = jnp.sum((d2 <= tau).astype(jnp.float32), axis=1, keepdims=True)
    tau, _ = jax.lax.fori_loop(0, 4, backoff, (tau, cnt0))

    cntb = jnp.sum((d2 < tau).astype(jnp.float32), axis=1, keepdims=True)
    iota = jax.lax.broadcasted_iota(jnp.int32, (1, N), 1).astype(jnp.float32)
    jsel = jnp.where(d2 == tau, iota, jnp.inf)

    def tie_body(_, carry):
        jthr, cntj = carry
        jnext = jnp.min(jnp.where(jsel > jthr, jsel, jnp.inf), axis=1,
                        keepdims=True)
        move = cntj < K
        return (jnp.where(move, jnext, jthr),
                cntj + move.astype(jnp.float32))

    jthr0 = jnp.full((R, 1), -jnp.inf, dtype=jnp.float32)
    jthr, cntj = jax.lax.fori_loop(0, 4, tie_body, (jthr0, cntb))
    keep_all = cntj < K
    A = (jnp.logical_or(d2 < tau,
                        jnp.logical_or(jsel <= jthr,
                                       jnp.logical_and(d2 == tau, keep_all)))
         ).astype(jnp.float32)

    nm = jnp.dot(A, hwe_ref[0], precision=jax.lax.Precision.HIGHEST)
    hp = nm[:, : H * D] / nm[:, H * D:]
    y = hp + x_ref[0]
    mu = jnp.mean(y, axis=1, keepdims=True)
    yc = y - mu
    var = jnp.mean(yc * yc, axis=1, keepdims=True)
    out = yc * jax.lax.rsqrt(var + 1e-5) * lns_ref[...] + lnb_ref[...]
    out_ref[0] = out


def kernel(x, positions, W, a_src, a_dst, ln_scale, ln_bias, topk):
    HD = H * D
    a_flat = (a_src + a_dst).reshape(HD)
    grp = jnp.arange(HD, dtype=jnp.int32) // D
    heads = jnp.arange(H, dtype=jnp.int32)
    asel = jnp.where(grp[:, None] == heads[None, :], a_flat[:, None], 0.0)
    eind = (grp[None, :] == heads[:, None]).astype(jnp.float32)
    posT = jnp.transpose(positions, (0, 2, 1))
    p2row = ((positions ** 2).sum(axis=-1)).reshape(B, 1, N)
    lns = ln_scale.reshape(1, HD)
    lnb = ln_bias.reshape(1, HD)

    hwe = pl.pallas_call(
        _prologue_kernel,
        grid=(B,),
        in_specs=[
            pl.BlockSpec((1, N, IN_F), lambda b: (b, 0, 0)),
            pl.BlockSpec((IN_F, HD), lambda b: (0, 0)),
            pl.BlockSpec((HD, H), lambda b: (0, 0)),
            pl.BlockSpec((H, HD), lambda b: (0, 0)),
        ],
        out_specs=pl.BlockSpec((1, N, 2 * HD), lambda b: (b, 0, 0)),
        out_shape=jax.ShapeDtypeStruct((B, N, 2 * HD), jnp.float32),
        compiler_params=pltpu.CompilerParams(
            dimension_semantics=("parallel",)),
    )(x, W, asel, eind)

    out = pl.pallas_call(
        _main_kernel,
        grid=(B, N // R),
        in_specs=[
            pl.BlockSpec((1, R, 3), lambda b, i: (b, i, 0)),
            pl.BlockSpec((1, 3, N), lambda b, i: (b, 0, 0)),
            pl.BlockSpec((1, 1, N), lambda b, i: (b, 0, 0)),
            pl.BlockSpec((1, N, 2 * HD), lambda b, i: (b, 0, 0)),
            pl.BlockSpec((1, R, IN_F), lambda b, i: (b, i, 0)),
            pl.BlockSpec((1, HD), lambda b, i: (0, 0)),
            pl.BlockSpec((1, HD), lambda b, i: (0, 0)),
        ],
        out_specs=pl.BlockSpec((1, R, IN_F), lambda b, i: (b, i, 0)),
        out_shape=jax.ShapeDtypeStruct((B, N, IN_F), jnp.float32),
        compiler_params=pltpu.CompilerParams(
            dimension_semantics=("parallel", "parallel")),
    )(positions, posT, p2row, hwe, x, lns, lnb)
    return out


# speedup vs baseline: 1.0500x; 1.0021x over previous
import jax
import jax.numpy as jnp
from jax.experimental import pallas as pl
from jax.experimental.pallas import tpu as pltpu

B, N, IN_F = 2, 4096, 128
H, D = 4, 32
K = 32
R = 512


def _prologue_kernel(x_ref, w_ref, asel_ref, eind_ref, hwe_ref):
    x = x_ref[0]
    W = w_ref[...]
    h = jnp.dot(x.astype(jnp.bfloat16), W.astype(jnp.bfloat16),
                preferred_element_type=jnp.float32)
    e = jnp.dot(h, asel_ref[...], precision=jax.lax.Precision.HIGHEST)
    f = jnp.where(e >= 0.0, e, 0.2 * e)
    fmax = jnp.max(f, axis=0, keepdims=True)
    expf = jnp.exp(f - fmax)
    expfb = jnp.dot(expf, eind_ref[...],
                    precision=jax.lax.Precision.HIGHEST)
    hwe_ref[0, :, : H * D] = h * expfb
    hwe_ref[0, :, H * D:] = expfb


def _main_kernel(pos_ref, post_ref, p2row_ref, hwe_ref, x_ref, lns_ref,
                 lnb_ref, out_ref):
    pos = pos_ref[0]
    posT = post_ref[0]
    dot = jnp.dot(pos.astype(jnp.bfloat16), posT.astype(jnp.bfloat16),
                  preferred_element_type=jnp.float32)
    p2b = jnp.sum(pos * pos, axis=1, keepdims=True)
    p2f = p2row_ref[0]
    d2 = p2b + p2f - 2.0 * dot

    def body(_, m):
        return jnp.min(jnp.where(d2 > m, d2, jnp.inf), axis=1, keepdims=True)

    m0 = jnp.full((R, 1), -jnp.inf, dtype=jnp.float32)
    tau = jax.lax.fori_loop(0, K, body, m0, unroll=8)

    def backoff(_, carry):
        tau, cnt = carry
        prev = jnp.max(jnp.where(d2 < tau, d2, -jnp.inf), axis=1,
                       keepdims=True)
        cntp = jnp.sum((d2 <= prev).astype(jnp.float32), axis=1,
                       keepdims=True)
        move = jnp.logical_and(cnt > K, cntp >= K)
        return jnp.where(move, prev, tau), jnp.where(move, cntp, cnt)

    cnt0 = jnp.sum((d2 <= tau).astype(jnp.float32), axis=1, keepdims=True)
    tau, _ = jax.lax.fori_loop(0, 4, backoff, (tau, cnt0), unroll=4)

    cntb = jnp.sum((d2 < tau).astype(jnp.float32), axis=1, keepdims=True)
    iota = jax.lax.broadcasted_iota(jnp.int32, (1, N), 1).astype(jnp.float32)
    jsel = jnp.where(d2 == tau, iota, jnp.inf)

    def tie_body(_, carry):
        jthr, cntj = carry
        jnext = jnp.min(jnp.where(jsel > jthr, jsel, jnp.inf), axis=1,
                        keepdims=True)
        move = cntj < K
        return (jnp.where(move, jnext, jthr),
                cntj + move.astype(jnp.float32))

    jthr0 = jnp.full((R, 1), -jnp.inf, dtype=jnp.float32)
    jthr, cntj = jax.lax.fori_loop(0, 4, tie_body, (jthr0, cntb), unroll=4)
    keep_all = cntj < K
    A = (jnp.logical_or(d2 < tau,
                        jnp.logical_or(jsel <= jthr,
                                       jnp.logical_and(d2 == tau, keep_all)))
         ).astype(jnp.float32)

    nm = jnp.dot(A, hwe_ref[0], precision=jax.lax.Precision.HIGHEST)
    hp = nm[:, : H * D] / nm[:, H * D:]
    y = hp + x_ref[0]
    mu = jnp.mean(y, axis=1, keepdims=True)
    yc = y - mu
    var = jnp.mean(yc * yc, axis=1, keepdims=True)
    out = yc * jax.lax.rsqrt(var + 1e-5) * lns_ref[...] + lnb_ref[...]
    out_ref[0] = out


def kernel(x, positions, W, a_src, a_dst, ln_scale, ln_bias, topk):
    HD = H * D
    a_flat = (a_src + a_dst).reshape(HD)
    grp = jnp.arange(HD, dtype=jnp.int32) // D
    heads = jnp.arange(H, dtype=jnp.int32)
    asel = jnp.where(grp[:, None] == heads[None, :], a_flat[:, None], 0.0)
    eind = (grp[None, :] == heads[:, None]).astype(jnp.float32)
    posT = jnp.transpose(positions, (0, 2, 1))
    p2row = ((positions ** 2).sum(axis=-1)).reshape(B, 1, N)
    lns = ln_scale.reshape(1, HD)
    lnb = ln_bias.reshape(1, HD)

    hwe = pl.pallas_call(
        _prologue_kernel,
        grid=(B,),
        in_specs=[
            pl.BlockSpec((1, N, IN_F), lambda b: (b, 0, 0)),
            pl.BlockSpec((IN_F, HD), lambda b: (0, 0)),
            pl.BlockSpec((HD, H), lambda b: (0, 0)),
            pl.BlockSpec((H, HD), lambda b: (0, 0)),
        ],
        out_specs=pl.BlockSpec((1, N, 2 * HD), lambda b: (b, 0, 0)),
        out_shape=jax.ShapeDtypeStruct((B, N, 2 * HD), jnp.float32),
        compiler_params=pltpu.CompilerParams(
            dimension_semantics=("parallel",)),
    )(x, W, asel, eind)

    out = pl.pallas_call(
        _main_kernel,
        grid=(B, N // R),
        in_specs=[
            pl.BlockSpec((1, R, 3), lambda b, i: (b, i, 0)),
            pl.BlockSpec((1, 3, N), lambda b, i: (b, 0, 0)),
            pl.BlockSpec((1, 1, N), lambda b, i: (b, 0, 0)),
            pl.BlockSpec((1, N, 2 * HD), lambda b, i: (b, 0, 0)),
            pl.BlockSpec((1, R, IN_F), lambda b, i: (b, i, 0)),
            pl.BlockSpec((1, HD), lambda b, i: (0, 0)),
            pl.BlockSpec((1, HD), lambda b, i: (0, 0)),
        ],
        out_specs=pl.BlockSpec((1, R, IN_F), lambda b, i: (b, i, 0)),
        out_shape=jax.ShapeDtypeStruct((B, N, IN_F), jnp.float32),
        compiler_params=pltpu.CompilerParams(
            dimension_semantics=("parallel", "parallel")),
    )(positions, posT, p2row, hwe, x, lns, lnb)
    return out
